# baseline (speedup 1.0000x reference)
import functools

import jax
import jax.numpy as jnp
from jax import lax
from jax.experimental import pallas as pl
from jax.experimental.pallas import tpu as pltpu
from jax.experimental.pallas import tpu_sc as plsc

B, F, V, D = 16384, 26, 100000, 16
DP = D + 1
BC = 128
NG = BC // 16
DRAIN_LAG = 4


def _fm_body(tab_hbm, idx_hbm, coef_hbm, w0_hbm,
             score_hbm, emb_hbm,
             idx_v, coef_v, emb_v, first_v, score_v, w0_v,
             sem, nc, rows_per_w):
    wid = lax.axis_index("s") * nc + lax.axis_index("c")
    nchunk = rows_per_w // BC

    pltpu.sync_copy(w0_hbm, w0_v.at[pl.ds(0, 1)])
    w0s = w0_v[pl.ds(0, 16)][0]

    def chunk_body(ci, _):
        b0 = wid * rows_per_w + ci * BC
        pltpu.sync_copy(idx_hbm.at[:, pl.ds(b0, BC)], idx_v)
        pltpu.sync_copy(coef_hbm.at[:, pl.ds(b0, BC)], coef_v)

        cps = []
        for f in range(F):
            row = idx_v.at[f]
            for d in range(D):
                cps.append(pltpu.async_copy(
                    tab_hbm.at[f].at[d].at[row], emb_v.at[f, d], sem))
            cps.append(pltpu.async_copy(
                tab_hbm.at[f].at[D].at[row], first_v.at[f], sem))
            if f >= DRAIN_LAG:
                for cp in cps[(f - DRAIN_LAG) * DP:(f - DRAIN_LAG + 1) * DP]:
                    cp.wait()
        for cp in cps[(F - DRAIN_LAG) * DP:]:
            cp.wait()

        def group_body(g, _):
            zero = jnp.zeros((16,), jnp.float32)
            tmpd = [zero] * D
            acc2 = zero
            fs = zero
            for f in range(F):
                cf = coef_v[f, pl.ds(g * 16, 16)]
                for d in range(D):
                    e = emb_v[f, d, pl.ds(g * 16, 16)]
                    t = cf * e
                    tmpd[d] = tmpd[d] + t
                    acc2 = acc2 + t * t
                fs = fs + cf * first_v[f, pl.ds(g * 16, 16)]
            sq = zero
            for d in range(D):
                sq = sq + tmpd[d] * tmpd[d]
            score_v[pl.ds(g * 16, 16)] = w0s + fs + 0.5 * (sq - acc2)
            return 0
        lax.fori_loop(0, NG, group_body, 0)

        pltpu.sync_copy(emb_v, emb_hbm.at[:, :, pl.ds(b0, BC)])
        pltpu.sync_copy(score_v, score_hbm.at[pl.ds(b0, BC)])
        return 0

    lax.fori_loop(0, nchunk, chunk_body, 0)


def kernel(index, coef, w0, first_tables, second_tables):
    info = plsc.get_sparse_core_info()
    nc, ns = info.num_cores, info.num_subcores
    nw = nc * ns
    rows_per_w = B // nw

    tab = jnp.concatenate(
        [jnp.transpose(second_tables, (0, 2, 1)),
         jnp.transpose(first_tables, (0, 2, 1))], axis=1)
    idx_t = index.T
    coef_t = coef.T

    mesh = plsc.VectorSubcoreMesh(core_axis_name="c", subcore_axis_name="s")
    body = functools.partial(_fm_body, nc=nc, rows_per_w=rows_per_w)
    fn = pl.kernel(
        body,
        out_type=(jax.ShapeDtypeStruct((B,), jnp.float32),
                  jax.ShapeDtypeStruct((F, D, B), jnp.float32)),
        mesh=mesh,
        compiler_params=pltpu.CompilerParams(needs_layout_passes=False,
                                             use_tc_tiling_on_sc=False),
        scratch_types=[
            pltpu.VMEM((F, BC), jnp.int32),
            pltpu.VMEM((F, BC), jnp.float32),
            pltpu.VMEM((F, D, BC), jnp.float32),
            pltpu.VMEM((F, BC), jnp.float32),
            pltpu.VMEM((BC,), jnp.float32),
            pltpu.VMEM((16,), jnp.float32),
            pltpu.SemaphoreType.DMA,
        ],
    )
    score, emb_t = fn(tab, idx_t, coef_t, w0)
    return score, jnp.transpose(emb_t, (2, 0, 1))


# speedup vs baseline: 3.4033x; 3.4033x over previous
import functools

import jax
import jax.numpy as jnp
from jax import lax
from jax.experimental import pallas as pl
from jax.experimental.pallas import tpu as pltpu
from jax.experimental.pallas import tpu_sc as plsc

B, F, V, D = 16384, 26, 100000, 16
DP = D + 1
BC = 128
NG = BC // 16
DRAIN_LAG = 4


def _fm_body(tab_hbm, ft_hbm, idx_hbm, coef_hbm, w0_hbm,
             score_hbm, emb_hbm,
             idx_v, coef_v, emb_v, first_v, score_v, w0_v,
             sem, nc, rows_per_w):
    wid = lax.axis_index("s") * nc + lax.axis_index("c")
    nchunk = rows_per_w // BC

    pltpu.sync_copy(w0_hbm, w0_v.at[pl.ds(0, 1)])
    w0s = w0_v[pl.ds(0, 16)][0]

    def chunk_body(ci, _):
        b0 = wid * rows_per_w + ci * BC
        pltpu.sync_copy(idx_hbm.at[:, pl.ds(b0, BC)], idx_v)
        pltpu.sync_copy(coef_hbm.at[:, pl.ds(b0, BC)], coef_v)

        cps = []
        for f in range(F):
            row = idx_v.at[f]
            for d in range(D):
                cps.append(pltpu.async_copy(
                    tab_hbm.at[f].at[d].at[row], emb_v.at[f, d], sem))
            cps.append(pltpu.async_copy(
                ft_hbm.at[f].at[row], first_v.at[f], sem))
            if f >= DRAIN_LAG:
                for cp in cps[(f - DRAIN_LAG) * DP:(f - DRAIN_LAG + 1) * DP]:
                    cp.wait()
        for cp in cps[(F - DRAIN_LAG) * DP:]:
            cp.wait()

        def group_body(g, _):
            zero = jnp.zeros((16,), jnp.float32)
            tmpd = [zero] * D
            acc2 = zero
            fs = zero
            for f in range(F):
                cf = coef_v[f, pl.ds(g * 16, 16)]
                for d in range(D):
                    e = emb_v[f, d, pl.ds(g * 16, 16)]
                    t = cf * e
                    tmpd[d] = tmpd[d] + t
                    acc2 = acc2 + t * t
                fs = fs + cf * first_v[f, pl.ds(g * 16, 16)]
            sq = zero
            for d in range(D):
                sq = sq + tmpd[d] * tmpd[d]
            score_v[pl.ds(g * 16, 16)] = w0s + fs + 0.5 * (sq - acc2)
            return 0
        lax.fori_loop(0, NG, group_body, 0)

        pltpu.sync_copy(emb_v, emb_hbm.at[:, :, pl.ds(b0, BC)])
        pltpu.sync_copy(score_v, score_hbm.at[pl.ds(b0, BC)])
        return 0

    lax.fori_loop(0, nchunk, chunk_body, 0)


def kernel(index, coef, w0, first_tables, second_tables):
    info = plsc.get_sparse_core_info()
    nc, ns = info.num_cores, info.num_subcores
    nw = nc * ns
    rows_per_w = B // nw

    tab = jnp.transpose(second_tables, (0, 2, 1))
    ft = first_tables[:, :, 0]
    idx_t = index.T
    coef_t = coef.T

    mesh = plsc.VectorSubcoreMesh(core_axis_name="c", subcore_axis_name="s")
    body = functools.partial(_fm_body, nc=nc, rows_per_w=rows_per_w)
    fn = pl.kernel(
        body,
        out_type=(jax.ShapeDtypeStruct((B,), jnp.float32),
                  jax.ShapeDtypeStruct((F, D, B), jnp.float32)),
        mesh=mesh,
        compiler_params=pltpu.CompilerParams(needs_layout_passes=False,
                                             use_tc_tiling_on_sc=False),
        scratch_types=[
            pltpu.VMEM((F, BC), jnp.int32),
            pltpu.VMEM((F, BC), jnp.float32),
            pltpu.VMEM((F, D, BC), jnp.float32),
            pltpu.VMEM((F, BC), jnp.float32),
            pltpu.VMEM((BC,), jnp.float32),
            pltpu.VMEM((16,), jnp.float32),
            pltpu.SemaphoreType.DMA,
        ],
    )
    score, emb_t = fn(tab, ft, idx_t, coef_t, w0)
    return score, jnp.transpose(emb_t, (2, 0, 1))


# speedup vs baseline: 3.5560x; 1.0449x over previous
import functools

import jax
import jax.numpy as jnp
from jax import lax
from jax.experimental import pallas as pl
from jax.experimental.pallas import tpu as pltpu
from jax.experimental.pallas import tpu_sc as plsc

B, F, V, D = 16384, 26, 100000, 16
DP = D + 1
BC = 128
NG = BC // 16


def _fm_body(tab_hbm, ft_hbm, idx_hbm, coef_hbm, w0_hbm,
             score_hbm, emb_hbm,
             idx_v, coef_v, emb_v, first_v, acc1_v, acc2_v, fs_v,
             score_v, w0_v, sem, nc, rows_per_w):
    wid = lax.axis_index("s") * nc + lax.axis_index("c")
    nchunk = rows_per_w // BC
    zero = jnp.zeros((16,), jnp.float32)

    pltpu.sync_copy(w0_hbm, w0_v.at[pl.ds(0, 1)])
    w0s = w0_v[pl.ds(0, 16)][0]

    def chunk_body(ci, _):
        b0 = wid * rows_per_w + ci * BC
        pltpu.sync_copy(idx_hbm.at[:, pl.ds(b0, BC)], idx_v)
        pltpu.sync_copy(coef_hbm.at[:, pl.ds(b0, BC)], coef_v)

        cps = []
        for f in range(F):
            row = idx_v.at[f]
            for d in range(D):
                cps.append(pltpu.async_copy(
                    tab_hbm.at[f].at[d].at[row], emb_v.at[f, d], sem))
            cps.append(pltpu.async_copy(
                ft_hbm.at[f].at[row], first_v.at[f], sem))

        def zero_body(g, _):
            gs = pl.ds(g * 16, 16)
            for d in range(D):
                acc1_v[d, gs] = zero
            acc2_v[gs] = zero
            fs_v[gs] = zero
            return 0
        lax.fori_loop(0, NG, zero_body, 0)

        for f in range(F):
            for cp in cps[f * DP:(f + 1) * DP]:
                cp.wait()

            def fold_body(g, _, f=f):
                gs = pl.ds(g * 16, 16)
                cf = coef_v[f, gs]
                t2 = zero
                for d in range(D):
                    t = cf * emb_v[f, d, gs]
                    acc1_v[d, gs] = acc1_v[d, gs] + t
                    t2 = t2 + t * t
                acc2_v[gs] = acc2_v[gs] + t2
                fs_v[gs] = fs_v[gs] + cf * first_v[f, gs]
                return 0
            lax.fori_loop(0, NG, fold_body, 0)

        def score_body(g, _):
            gs = pl.ds(g * 16, 16)
            sq = zero
            for d in range(D):
                a = acc1_v[d, gs]
                sq = sq + a * a
            score_v[gs] = w0s + fs_v[gs] + 0.5 * (sq - acc2_v[gs])
            return 0
        lax.fori_loop(0, NG, score_body, 0)

        pltpu.sync_copy(emb_v, emb_hbm.at[:, :, pl.ds(b0, BC)])
        pltpu.sync_copy(score_v, score_hbm.at[pl.ds(b0, BC)])
        return 0

    lax.fori_loop(0, nchunk, chunk_body, 0)


def kernel(index, coef, w0, first_tables, second_tables):
    info = plsc.get_sparse_core_info()
    nc, ns = info.num_cores, info.num_subcores
    nw = nc * ns
    rows_per_w = B // nw

    tab = jnp.transpose(second_tables, (0, 2, 1))
    ft = first_tables.reshape(F, V)
    idx_t = index.T
    coef_t = coef.T

    mesh = plsc.VectorSubcoreMesh(core_axis_name="c", subcore_axis_name="s")
    body = functools.partial(_fm_body, nc=nc, rows_per_w=rows_per_w)
    fn = pl.kernel(
        body,
        out_type=(jax.ShapeDtypeStruct((B,), jnp.float32),
                  jax.ShapeDtypeStruct((F, D, B), jnp.float32)),
        mesh=mesh,
        compiler_params=pltpu.CompilerParams(needs_layout_passes=False,
                                             use_tc_tiling_on_sc=False),
        scratch_types=[
            pltpu.VMEM((F, BC), jnp.int32),
            pltpu.VMEM((F, BC), jnp.float32),
            pltpu.VMEM((F, D, BC), jnp.float32),
            pltpu.VMEM((F, BC), jnp.float32),
            pltpu.VMEM((D, BC), jnp.float32),
            pltpu.VMEM((BC,), jnp.float32),
            pltpu.VMEM((BC,), jnp.float32),
            pltpu.VMEM((BC,), jnp.float32),
            pltpu.VMEM((16,), jnp.float32),
            pltpu.SemaphoreType.DMA,
        ],
    )
    score, emb_t = fn(tab, ft, idx_t, coef_t, w0)
    return score, jnp.transpose(emb_t, (2, 0, 1))


# speedup vs baseline: 3.6283x; 1.0203x over previous
import functools

import jax
import jax.numpy as jnp
from jax import lax
from jax.experimental import pallas as pl
from jax.experimental.pallas import tpu as pltpu
from jax.experimental.pallas import tpu_sc as plsc

B, F, V, D = 16384, 26, 100000, 16
DP = D + 1
BC = 128
NG = BC // 16


def _fm_body(tab_hbm, ft_hbm, idx_hbm, coef_hbm, w0_hbm,
             score_hbm, emb_hbm,
             idx_v, coef_v, emb_v, first_v, acc1_v, acc2_v, fs_v,
             score_v, w0_v, sem, nc, rows_per_w):
    wid = lax.axis_index("s") * nc + lax.axis_index("c")
    nchunk = rows_per_w // BC
    zero = jnp.zeros((16,), jnp.float32)

    pltpu.sync_copy(w0_hbm, w0_v.at[pl.ds(0, 1)])
    w0s = w0_v[pl.ds(0, 16)][0]

    def chunk_body(ci, _):
        b0 = wid * rows_per_w + ci * BC
        pltpu.sync_copy(idx_hbm.at[:, pl.ds(b0, BC)], idx_v)
        pltpu.sync_copy(coef_hbm.at[:, pl.ds(b0, BC)], coef_v)

        cps = []
        for f in range(F):
            row = idx_v.at[f]
            for d in range(D):
                cps.append(pltpu.async_copy(
                    tab_hbm.at[f].at[d].at[row], emb_v.at[f, d], sem))
            cps.append(pltpu.async_copy(
                ft_hbm.at[f].at[0].at[row], first_v.at[f], sem))

        def zero_body(g, _):
            gs = pl.ds(g * 16, 16)
            for d in range(D):
                acc1_v[d, gs] = zero
            acc2_v[gs] = zero
            fs_v[gs] = zero
            return 0
        lax.fori_loop(0, NG, zero_body, 0)

        for f in range(F):
            for cp in cps[f * DP:(f + 1) * DP]:
                cp.wait()

            def fold_body(g, _, f=f):
                gs = pl.ds(g * 16, 16)
                cf = coef_v[f, gs]
                t2 = zero
                for d in range(D):
                    t = cf * emb_v[f, d, gs]
                    acc1_v[d, gs] = acc1_v[d, gs] + t
                    t2 = t2 + t * t
                acc2_v[gs] = acc2_v[gs] + t2
                fs_v[gs] = fs_v[gs] + cf * first_v[f, gs]
                return 0
            lax.fori_loop(0, NG, fold_body, 0)

        def score_body(g, _):
            gs = pl.ds(g * 16, 16)
            sq = zero
            for d in range(D):
                a = acc1_v[d, gs]
                sq = sq + a * a
            score_v[gs] = w0s + fs_v[gs] + 0.5 * (sq - acc2_v[gs])
            return 0
        lax.fori_loop(0, NG, score_body, 0)

        pltpu.sync_copy(emb_v, emb_hbm.at[:, :, pl.ds(b0, BC)])
        pltpu.sync_copy(score_v, score_hbm.at[pl.ds(b0, BC)])
        return 0

    lax.fori_loop(0, nchunk, chunk_body, 0)


def kernel(index, coef, w0, first_tables, second_tables):
    info = plsc.get_sparse_core_info()
    nc, ns = info.num_cores, info.num_subcores
    nw = nc * ns
    rows_per_w = B // nw

    tab = jnp.transpose(second_tables, (0, 2, 1))
    ft = jnp.transpose(first_tables, (0, 2, 1))
    idx_t = index.T
    coef_t = coef.T

    mesh = plsc.VectorSubcoreMesh(core_axis_name="c", subcore_axis_name="s")
    body = functools.partial(_fm_body, nc=nc, rows_per_w=rows_per_w)
    fn = pl.kernel(
        body,
        out_type=(jax.ShapeDtypeStruct((B,), jnp.float32),
                  jax.ShapeDtypeStruct((F, D, B), jnp.float32)),
        mesh=mesh,
        compiler_params=pltpu.CompilerParams(needs_layout_passes=False,
                                             use_tc_tiling_on_sc=False),
        scratch_types=[
            pltpu.VMEM((F, BC), jnp.int32),
            pltpu.VMEM((F, BC), jnp.float32),
            pltpu.VMEM((F, D, BC), jnp.float32),
            pltpu.VMEM((F, BC), jnp.float32),
            pltpu.VMEM((D, BC), jnp.float32),
            pltpu.VMEM((BC,), jnp.float32),
            pltpu.VMEM((BC,), jnp.float32),
            pltpu.VMEM((BC,), jnp.float32),
            pltpu.VMEM((16,), jnp.float32),
            pltpu.SemaphoreType.DMA,
        ],
    )
    score, emb_t = fn(tab, ft, idx_t, coef_t, w0)
    return score, jnp.transpose(emb_t, (2, 0, 1))
